# baseline (speedup 1.0000x reference)
import functools

import jax
import jax.numpy as jnp
from jax import lax
from jax.experimental import pallas as pl
from jax.experimental.pallas import tpu as pltpu
from jax.experimental.pallas import tpu_sc as plsc

N = 1_000_000
H = 32
NP = 1 << 20
NW = 32
PER_W = NP // NW
C = 8192
G = PER_W // C
L = 16


def _axon_body(hist_hbm, ap_hbm, delay_hbm, out_hbm,
               delay_v, ap_v, idx_v, gath_v, sem):
    c = lax.axis_index("c")
    s = lax.axis_index("s")
    wid = s * 2 + c
    lane = lax.iota(jnp.int32, L)

    for g in range(G):
        base = wid * PER_W + g * C
        pltpu.sync_copy(delay_hbm.at[pl.ds(base, C)], delay_v)
        pltpu.sync_copy(ap_hbm.at[pl.ds(base, C)], ap_v)

        def idx_body(j, pos):
            off = j * L
            d = delay_v[pl.ds(off, L)]
            dm1 = jnp.maximum(d - 1, 0)
            idx_v[pl.ds(off, L)] = dm1 * N + pos
            return pos + L

        lax.fori_loop(0, C // L, idx_body, base + lane)

        pltpu.async_copy(hist_hbm.at[idx_v], gath_v, sem).wait()

        def sel_body(j, carry):
            off = j * L
            d = delay_v[pl.ds(off, L)]
            g_ = gath_v[pl.ds(off, L)]
            a = ap_v[pl.ds(off, L)]
            gath_v[pl.ds(off, L)] = jnp.where(d == 0, a, g_)
            return carry

        lax.fori_loop(0, C // L, sel_body, 0)

        pltpu.sync_copy(gath_v, out_hbm.at[pl.ds(base, C)])


@jax.jit
def _axon(hist_flat, ap_p, delay_p):
    mesh = plsc.VectorSubcoreMesh(core_axis_name="c", subcore_axis_name="s")
    return pl.kernel(
        _axon_body,
        out_type=jax.ShapeDtypeStruct((NP,), jnp.float32),
        mesh=mesh,
        scratch_types=[
            pltpu.VMEM((C,), jnp.int32),
            pltpu.VMEM((C,), jnp.float32),
            pltpu.VMEM((C,), jnp.int32),
            pltpu.VMEM((C,), jnp.float32),
            pltpu.SemaphoreType.DMA,
        ],
    )(hist_flat, ap_p, delay_p)


def kernel(action_potential, history, delay):
    pad = NP - N
    ap_p = jnp.concatenate([action_potential,
                            jnp.zeros((pad,), jnp.float32)])
    delay_p = jnp.concatenate([delay.astype(jnp.int32),
                               jnp.zeros((pad,), jnp.int32)])
    hist_flat = history.reshape(-1)
    out = _axon(hist_flat, ap_p, delay_p)
    return out[:N]
